# baseline (speedup 1.0000x reference)
import jax
import jax.numpy as jnp
from jax.experimental import pallas as pl
from jax.experimental.pallas import tpu as pltpu

N = 4096
D = 128
BLK = 512
NBLK = N // BLK


def _koleo_kernel(emb_blk_ref, emb_ref, out_ref, rowmin_ref, gmax_ref, sqn_ref):
    i = pl.program_id(0)
    emb = emb_ref[...]
    blk = emb_blk_ref[...]

    @pl.when(i == 0)
    def _():
        sqn_ref[...] = jnp.sum(emb * emb, axis=1)[None, :]

    sqn_all = sqn_ref[...]
    sqn_blk = jnp.sum(blk * blk, axis=1)[:, None]

    dot2 = jax.lax.dot_general(
        blk + blk, emb, (((1,), (1,)), ((), ())),
        preferred_element_type=jnp.float32)

    sq = sqn_blk + sqn_all - dot2
    masked = jnp.where(sq <= 0.0, jnp.inf, sq)
    rowmin = jnp.min(masked, axis=1)[:, None]
    tilemax = jnp.max(sq)

    rowmin_ref[pl.ds(i, 1), :] = rowmin.reshape(1, BLK)

    @pl.when(i == 0)
    def _():
        gmax_ref[0, 0] = tilemax

    @pl.when(i > 0)
    def _():
        gmax_ref[0, 0] = jnp.maximum(gmax_ref[0, 0], tilemax)

    @pl.when(i == NBLK - 1)
    def _():
        g = jnp.maximum(gmax_ref[0, 0], 0.0)
        m = jnp.minimum(rowmin_ref[...], g)
        d = jnp.sqrt(m)
        out_ref[...] = jnp.reshape((-1.0 / N) * jnp.sum(jnp.log(d)), (1, 1))


def kernel(embeddings):
    out = pl.pallas_call(
        _koleo_kernel,
        grid=(NBLK,),
        in_specs=[
            pl.BlockSpec((BLK, D), lambda i: (i, 0)),
            pl.BlockSpec((N, D), lambda i: (0, 0)),
        ],
        out_specs=pl.BlockSpec((1, 1), lambda i: (0, 0)),
        out_shape=jax.ShapeDtypeStruct((1, 1), jnp.float32),
        scratch_shapes=[
            pltpu.VMEM((NBLK, BLK), jnp.float32),
            pltpu.SMEM((1, 1), jnp.float32),
            pltpu.VMEM((1, N), jnp.float32),
        ],
    )(embeddings, embeddings)
    return out[0, 0]


# speedup vs baseline: 1.0100x; 1.0100x over previous
import jax
import jax.numpy as jnp
from jax.experimental import pallas as pl
from jax.experimental.pallas import tpu as pltpu

N = 4096
D = 128
BLK = 512
NBLK = N // BLK


def _main_kernel(emb_blk_ref, emb_ref, rowmin_ref, tmax_ref):
    emb = emb_ref[...]
    blk = emb_blk_ref[...]

    sqn_all = jnp.sum(emb * emb, axis=1)[None, :]
    sqn_blk = jnp.sum(blk * blk, axis=1)[:, None]

    dot2 = jax.lax.dot_general(
        blk + blk, emb, (((1,), (1,)), ((), ())),
        preferred_element_type=jnp.float32)

    sq = sqn_blk + sqn_all - dot2
    masked = jnp.where(sq <= 0.0, jnp.inf, sq)
    rowmin_ref[...] = jnp.min(masked, axis=1).reshape(1, 1, BLK)
    tmax_ref[...] = jnp.reshape(jnp.max(sq), (1, 1, 1))


def _epilogue_kernel(rowmin_ref, tmax_ref, out_ref):
    g = jnp.maximum(jnp.max(tmax_ref[...]), 0.0)
    m = jnp.minimum(rowmin_ref[...], g)
    d = jnp.sqrt(m)
    out_ref[...] = jnp.reshape((-1.0 / N) * jnp.sum(jnp.log(d)), (1, 1))


def kernel(embeddings):
    rowmin, tmax = pl.pallas_call(
        _main_kernel,
        grid=(NBLK,),
        in_specs=[
            pl.BlockSpec((BLK, D), lambda i: (i, 0)),
            pl.BlockSpec((N, D), lambda i: (0, 0)),
        ],
        out_specs=[
            pl.BlockSpec((1, 1, BLK), lambda i: (i, 0, 0)),
            pl.BlockSpec((1, 1, 1), lambda i: (i, 0, 0)),
        ],
        out_shape=[
            jax.ShapeDtypeStruct((NBLK, 1, BLK), jnp.float32),
            jax.ShapeDtypeStruct((NBLK, 1, 1), jnp.float32),
        ],
        compiler_params=pltpu.CompilerParams(
            dimension_semantics=("parallel",)),
    )(embeddings, embeddings)

    out = pl.pallas_call(
        _epilogue_kernel,
        out_shape=jax.ShapeDtypeStruct((1, 1), jnp.float32),
    )(rowmin, tmax)
    return out[0, 0]


# speedup vs baseline: 1.1600x; 1.1485x over previous
import numpy as np
import jax
import jax.numpy as jnp
from jax.experimental import pallas as pl
from jax.experimental.pallas import tpu as pltpu

N = 4096
D = 128
BLK = 512
NBLK = N // BLK
NTILES = NBLK * (NBLK + 1) // 2

_BI = np.array([bi for bi in range(NBLK) for bj in range(bi, NBLK)],
               dtype=np.int32)
_BJ = np.array([bj for bi in range(NBLK) for bj in range(bi, NBLK)],
               dtype=np.int32)


def _koleo_kernel(bi_ref, bj_ref, emb_ref, out_ref,
                  rmin_col_ref, rmin_row_ref, sqn_col_ref, sqn_row_ref,
                  gmax_ref):
    t = pl.program_id(0)
    bi = bi_ref[t]
    bj = bj_ref[t]

    @pl.when(t == 0)
    def _():
        emb = emb_ref[...]
        sqn = jnp.sum(emb * emb, axis=1)
        sqn_col_ref[...] = sqn[:, None]
        sqn_row_ref[...] = sqn[None, :]
        rmin_col_ref[...] = jnp.full((N, 1), jnp.inf, jnp.float32)
        rmin_row_ref[...] = jnp.full((1, N), jnp.inf, jnp.float32)
        gmax_ref[0, 0] = -jnp.inf

    blk_i = emb_ref[pl.ds(bi * BLK, BLK), :]
    blk_j = emb_ref[pl.ds(bj * BLK, BLK), :]
    sqn_i = sqn_col_ref[pl.ds(bi * BLK, BLK), :]
    sqn_j = sqn_row_ref[:, pl.ds(bj * BLK, BLK)]

    dot2 = jax.lax.dot_general(
        blk_i + blk_i, blk_j, (((1,), (1,)), ((), ())),
        preferred_element_type=jnp.float32)

    sq = sqn_i + sqn_j - dot2
    masked = jnp.where(sq <= 0.0, jnp.inf, sq)

    rmin_i = jnp.min(masked, axis=1)[:, None]
    rmin_j = jnp.min(masked, axis=0)[None, :]

    isl = pl.ds(bi * BLK, BLK)
    jsl = pl.ds(bj * BLK, BLK)
    rmin_col_ref[isl, :] = jnp.minimum(rmin_col_ref[isl, :], rmin_i)
    rmin_row_ref[:, jsl] = jnp.minimum(rmin_row_ref[:, jsl], rmin_j)
    gmax_ref[0, 0] = jnp.maximum(gmax_ref[0, 0], jnp.max(sq))

    @pl.when(t == NTILES - 1)
    def _():
        g = jnp.maximum(gmax_ref[0, 0], 0.0)
        a = rmin_col_ref[...].reshape(NBLK, BLK)
        b = rmin_row_ref[...].reshape(NBLK, BLK)
        m = jnp.minimum(jnp.minimum(a, b), g)
        d = jnp.sqrt(m)
        out_ref[...] = jnp.reshape((-1.0 / N) * jnp.sum(jnp.log(d)), (1, 1))


def kernel(embeddings):
    grid_spec = pltpu.PrefetchScalarGridSpec(
        num_scalar_prefetch=2,
        grid=(NTILES,),
        in_specs=[pl.BlockSpec((N, D), lambda t, bi, bj: (0, 0))],
        out_specs=pl.BlockSpec((1, 1), lambda t, bi, bj: (0, 0)),
        scratch_shapes=[
            pltpu.VMEM((N, 1), jnp.float32),
            pltpu.VMEM((1, N), jnp.float32),
            pltpu.VMEM((N, 1), jnp.float32),
            pltpu.VMEM((1, N), jnp.float32),
            pltpu.SMEM((1, 1), jnp.float32),
        ],
    )
    out = pl.pallas_call(
        _koleo_kernel,
        grid_spec=grid_spec,
        out_shape=jax.ShapeDtypeStruct((1, 1), jnp.float32),
    )(jnp.asarray(_BI), jnp.asarray(_BJ), embeddings)
    return out[0, 0]


# speedup vs baseline: 1.4391x; 1.2406x over previous
import numpy as np
import jax
import jax.numpy as jnp
from jax.experimental import pallas as pl
from jax.experimental.pallas import tpu as pltpu

N = 4096
D = 128
BLK = 1024
NBLK = N // BLK
NTILES = NBLK * (NBLK + 1) // 2

_BI = np.array([bi for bi in range(NBLK) for bj in range(bi, NBLK)],
               dtype=np.int32)
_BJ = np.array([bj for bi in range(NBLK) for bj in range(bi, NBLK)],
               dtype=np.int32)


def _koleo_kernel(bi_ref, bj_ref, emb_ref, out_ref,
                  rmin_col_ref, rmin_row_ref, sqn_col_ref, sqn_row_ref,
                  gmax_ref):
    t = pl.program_id(0)
    bi = bi_ref[t]
    bj = bj_ref[t]

    @pl.when(t == 0)
    def _():
        emb = emb_ref[...]
        sqn = jnp.sum(emb * emb, axis=1)
        sqn_col_ref[...] = sqn[:, None]
        sqn_row_ref[...] = sqn[None, :]
        rmin_col_ref[...] = jnp.full((N, 1), jnp.inf, jnp.float32)
        rmin_row_ref[...] = jnp.full((1, N), jnp.inf, jnp.float32)
        gmax_ref[0, 0] = -jnp.inf

    blk_i = emb_ref[pl.ds(bi * BLK, BLK), :]
    blk_j = emb_ref[pl.ds(bj * BLK, BLK), :]
    sqn_i = sqn_col_ref[pl.ds(bi * BLK, BLK), :]
    sqn_j = sqn_row_ref[:, pl.ds(bj * BLK, BLK)]

    dot2 = jax.lax.dot_general(
        blk_i + blk_i, blk_j, (((1,), (1,)), ((), ())),
        preferred_element_type=jnp.float32)

    sq = sqn_i + sqn_j - dot2
    masked = jnp.where(sq <= 0.0, jnp.inf, sq)

    rmin_i = jnp.min(masked, axis=1)[:, None]
    rmin_j = jnp.min(masked, axis=0)[None, :]

    isl = pl.ds(bi * BLK, BLK)
    jsl = pl.ds(bj * BLK, BLK)
    rmin_col_ref[isl, :] = jnp.minimum(rmin_col_ref[isl, :], rmin_i)
    rmin_row_ref[:, jsl] = jnp.minimum(rmin_row_ref[:, jsl], rmin_j)
    gmax_ref[0, 0] = jnp.maximum(gmax_ref[0, 0], jnp.max(sq))

    @pl.when(t == NTILES - 1)
    def _():
        g = jnp.maximum(gmax_ref[0, 0], 0.0)
        a = rmin_col_ref[...].reshape(NBLK, BLK)
        b = rmin_row_ref[...].reshape(NBLK, BLK)
        m = jnp.minimum(jnp.minimum(a, b), g)
        d = jnp.sqrt(m)
        out_ref[...] = jnp.reshape((-1.0 / N) * jnp.sum(jnp.log(d)), (1, 1))


def kernel(embeddings):
    grid_spec = pltpu.PrefetchScalarGridSpec(
        num_scalar_prefetch=2,
        grid=(NTILES,),
        in_specs=[pl.BlockSpec((N, D), lambda t, bi, bj: (0, 0))],
        out_specs=pl.BlockSpec((1, 1), lambda t, bi, bj: (0, 0)),
        scratch_shapes=[
            pltpu.VMEM((N, 1), jnp.float32),
            pltpu.VMEM((1, N), jnp.float32),
            pltpu.VMEM((N, 1), jnp.float32),
            pltpu.VMEM((1, N), jnp.float32),
            pltpu.SMEM((1, 1), jnp.float32),
        ],
    )
    out = pl.pallas_call(
        _koleo_kernel,
        grid_spec=grid_spec,
        out_shape=jax.ShapeDtypeStruct((1, 1), jnp.float32),
    )(jnp.asarray(_BI), jnp.asarray(_BJ), embeddings)
    return out[0, 0]
